# baseline (speedup 1.0000x reference)
import functools

import jax
import jax.numpy as jnp
from jax import lax
from jax.experimental import pallas as pl
from jax.experimental.pallas import tpu as pltpu
from jax.experimental.pallas import tpu_sc as plsc

GN = 3
NN = 10000
DD = 128
EE = 320000

NC = 2
NS = 16
NW = NC * NS
LN = 16

CHUNK = 40
CPB = 64
EPW = 256 * CHUNK
EPAD = EPW * NW
ROWS2D = EPAD // CHUNK
NSINK = 112
NA = NN + NSINK
RZ = NA // NS
RF = 624
RTAIL = NN - NS * RF

NBLK = 10
RB = NN // NBLK

_MESH = plsc.VectorSubcoreMesh(core_axis_name="c", subcore_axis_name="s",
                               num_cores=NC, num_subcores=NS)
_F32 = jnp.float32


@functools.partial(
    pl.kernel,
    out_type=jax.ShapeDtypeStruct((GN, NC, NN, DD), _F32),
    mesh=_MESH,
    scratch_types=[
        pltpu.VMEM_SHARED((NA, DD), _F32),
        pltpu.VMEM((CPB, CHUNK), jnp.int32),
        pltpu.VMEM((CHUNK, DD), _F32),
        pltpu.SemaphoreType.DMA,
    ],
)
def _deg(zeros_h, ones_h, d0, d1, d2, out, acc, didx, ones_b, ssem):
    c = lax.axis_index("c")
    s = lax.axis_index("s")
    w = s * NC + c
    pltpu.sync_copy(ones_h, ones_b)

    nblocks = EPW // CHUNK // CPB
    for g, dref in enumerate((d0, d1, d2)):
        pltpu.sync_copy(zeros_h.at[pl.ds(s * RZ, RZ)],
                        acc.at[pl.ds(s * RZ, RZ)])
        plsc.subcore_barrier()
        base = w * (EPW // CHUNK)

        @pl.loop(0, nblocks)
        def _(b):
            r0 = base + b * CPB
            pltpu.sync_copy(dref.at[pl.ds(r0, CPB)], didx)
            descs = [pltpu.async_copy(ones_b, acc.at[didx.at[j]], ssem,
                                      add=True)
                     for j in range(CPB)]
            for d in descs:
                d.wait()

        plsc.subcore_barrier()
        pltpu.sync_copy(acc.at[pl.ds(s * RF, RF)],
                        out.at[g, c, pl.ds(s * RF, RF)])

        @pl.when(s == NS - 1)
        def _():
            pltpu.sync_copy(acc.at[pl.ds(NS * RF, RTAIL)],
                            out.at[g, c, pl.ds(NS * RF, RTAIL)])

        plsc.subcore_barrier()


def _make_prop(kk, gids, off=0, ntab=None):
    if ntab is None:
        ntab = kk

    @functools.partial(
        pl.kernel,
        out_type=jax.ShapeDtypeStruct((kk, NC, NN, DD), _F32),
        mesh=_MESH,
        scratch_types=[
            pltpu.VMEM_SHARED((NA, DD), _F32),
            pltpu.VMEM((CPB, CHUNK), jnp.int32),
            pltpu.VMEM((CPB, CHUNK), jnp.int32),
            pltpu.VMEM((CHUNK, DD), _F32),
            pltpu.VMEM((CHUNK, DD), _F32),
            pltpu.VMEM((CHUNK, DD), _F32),
            pltpu.VMEM((CHUNK, DD), _F32),
            pltpu.VMEM((CHUNK, DD), _F32),
            pltpu.VMEM((CHUNK, DD), _F32),
            pltpu.SemaphoreType.DMA,
            pltpu.SemaphoreType.DMA,
            pltpu.SemaphoreType.DMA,
            pltpu.SemaphoreType.DMA,
            pltpu.SemaphoreType.DMA,
            pltpu.SemaphoreType.DMA,
            pltpu.SemaphoreType.DMA,
            pltpu.SemaphoreType.DMA,
            pltpu.SemaphoreType.DMA,
            pltpu.SemaphoreType.DMA,
            pltpu.SemaphoreType.DMA,
            pltpu.SemaphoreType.DMA,
        ],
    )
    def prop(zeros_h, tabs, s0, s1, s2, d0, d1, d2, out,
             acc, sidx, didx, r0b, r1b, r2b, r3b, r4b, r5b,
             gs0, gs1, gs2, gs3, gs4, gs5, ss0, ss1, ss2, ss3, ss4, ss5):
        c = lax.axis_index("c")
        s = lax.axis_index("s")
        w = s * NC + c

        rows = (r0b, r1b, r2b, r3b, r4b, r5b)
        gsem = (gs0, gs1, gs2, gs3, gs4, gs5)
        ssem = (ss0, ss1, ss2, ss3, ss4, ss5)
        srcs = (s0, s1, s2)
        dsts = (d0, d1, d2)
        nblocks = EPW // CHUNK // CPB
        for k in range(kk):
            g = gids[k]
            pltpu.sync_copy(zeros_h.at[pl.ds(s * RZ, RZ)],
                            acc.at[pl.ds(s * RZ, RZ)])
            plsc.subcore_barrier()
            base = w * (EPW // CHUNK)
            tab = tabs.at[off + k]

            @pl.loop(0, nblocks)
            def _(b):
                r0 = base + b * CPB
                pltpu.sync_copy(srcs[g].at[pl.ds(r0, CPB)], sidx)
                pltpu.sync_copy(dsts[g].at[pl.ds(r0, CPB)], didx)
                gd = [None] * CPB
                sd = [None] * CPB
                for j in range(3):
                    gd[j] = pltpu.async_copy(tab.at[sidx.at[j]], rows[j],
                                             gsem[j])
                for j in range(CPB):
                    p = j % 6
                    gd[j].wait()
                    sd[j] = pltpu.async_copy(rows[p], acc.at[didx.at[j]],
                                             ssem[p], add=True)
                    if j + 3 < CPB:
                        if j >= 3:
                            sd[j - 3].wait()
                        q = (j + 3) % 6
                        gd[j + 3] = pltpu.async_copy(
                            tab.at[sidx.at[j + 3]], rows[q], gsem[q])
                for j in range(CPB - 6, CPB):
                    sd[j].wait()

            plsc.subcore_barrier()
            pltpu.sync_copy(acc.at[pl.ds(s * RF, RF)],
                            out.at[k, c, pl.ds(s * RF, RF)])

            @pl.when(s == NS - 1)
            def _():
                pltpu.sync_copy(acc.at[pl.ds(NS * RF, RTAIL)],
                                out.at[k, c, pl.ds(NS * RF, RTAIL)])

            plsc.subcore_barrier()

    return prop


_prop3 = _make_prop(3, (0, 1, 2))
_prop3a = _make_prop(3, (0, 1, 2), off=0, ntab=6)
_prop3b = _make_prop(3, (0, 1, 2), off=3, ntab=6)


def _dot(a, b):
    return jnp.dot(a, b, preferred_element_type=_F32,
                   precision=lax.Precision.HIGHEST)


def _tc_dinv_body(deg_r, dinv_r):
    dd = deg_r[0]
    cnt = dd[0, :, 0:1] + dd[1, :, 0:1] + 1.0
    dinv = lax.rsqrt(jnp.maximum(cnt, 1e-12))
    dinv_r[0] = jnp.broadcast_to(dinv, (RB, LN))


def _tc_dinv(deg):
    return pl.pallas_call(
        _tc_dinv_body,
        grid=(GN, NBLK),
        in_specs=[pl.BlockSpec((1, NC, RB, DD), lambda g, rb: (g, 0, rb, 0))],
        out_specs=pl.BlockSpec((1, RB, LN), lambda g, rb: (g, rb, 0)),
        out_shape=jax.ShapeDtypeStruct((GN, NN, LN), _F32),
    )(deg)


def _tc_a_body(x0_r, x1_r, x2_r, w_r, b_r, wc_r, h_r, xw_r):
    wc = wc_r[...]
    for g, x_r in enumerate((x0_r, x1_r, x2_r)):
        h = _dot(x_r[...], w_r[g]) + b_r[g]
        xw_r[g] = _dot(h, wc)
        h_r[g] = h


_DINV_ALL = pl.BlockSpec((GN, RB, LN), lambda rb: (0, rb, 0))
_W_FULL = pl.BlockSpec((DD, DD), lambda rb: (0, 0))
_B_FULL = pl.BlockSpec((1, DD), lambda rb: (0, 0))


def _tc_a(x0, x1, x2, wfc1, bfc1, wc1):
    xspec = pl.BlockSpec((RB, DD), lambda rb: (rb, 0))
    g3 = pl.BlockSpec((GN, RB, DD), lambda rb: (0, rb, 0))
    return pl.pallas_call(
        _tc_a_body,
        grid=(NBLK,),
        in_specs=[
            xspec, xspec, xspec,
            pl.BlockSpec((GN, DD, DD), lambda rb: (0, 0, 0)),
            pl.BlockSpec((GN, 1, DD), lambda rb: (0, 0, 0)),
            _W_FULL,
        ],
        out_specs=[g3, g3],
        out_shape=[
            jax.ShapeDtypeStruct((GN, NN, DD), _F32),
            jax.ShapeDtypeStruct((GN, NN, DD), _F32),
        ],
    )(x0, x1, x2, wfc1, bfc1, wc1)


def _tc_scale_body(xw_r, dinv_r, u_r):
    for g in range(GN):
        u_r[g] = xw_r[g] * dinv_r[g][:, 0:1]


def _tc_scale(xw, dinv):
    g3 = pl.BlockSpec((GN, RB, DD), lambda rb: (0, rb, 0))
    return pl.pallas_call(
        _tc_scale_body,
        grid=(NBLK,),
        in_specs=[g3, _DINV_ALL],
        out_specs=g3,
        out_shape=jax.ShapeDtypeStruct((GN, NN, DD), _F32),
    )(xw, dinv)


def _post3_body(s_r, u_r, dinv_r, b_r, w2_r, c_r, u2_r):
    w2 = w2_r[...]
    for g in range(GN):
        dinv = dinv_r[g][:, 0:1]
        cfull = (s_r[g, 0] + s_r[g, 1] + u_r[g]) * dinv + b_r[...]
        c_r[g] = cfull
        u2_r[g] = _dot(cfull, w2) * dinv


def _post3_mm(s_arr, u_arr, dinv, bias, w2):
    g3 = pl.BlockSpec((GN, RB, DD), lambda rb: (0, rb, 0))
    return pl.pallas_call(
        _post3_body,
        grid=(NBLK,),
        in_specs=[
            pl.BlockSpec((GN, NC, RB, DD), lambda rb: (0, 0, rb, 0)),
            g3, _DINV_ALL, _B_FULL, _W_FULL,
        ],
        out_specs=[g3, g3],
        out_shape=[
            jax.ShapeDtypeStruct((GN, NN, DD), _F32),
            jax.ShapeDtypeStruct((GN, NN, DD), _F32),
        ],
    )(s_arr, u_arr, dinv, bias, w2)


def _tc_cd_body(s_r, u_r, dinv_r, b_r, wd1_r, c_r, uab_r, used_r, csp_r):
    wd1 = wd1_r[...]
    cs = []
    dinvs = []
    for g in range(GN):
        dinv = dinv_r[g][:, 0:1]
        dinvs.append(dinv)
        cfull = (s_r[g, 0] + s_r[g, 1] + u_r[g]) * dinv + b_r[...]
        c_r[g] = cfull
        cs.append(cfull)
        csp_r[0, g] = jnp.broadcast_to(
            jnp.sum(cfull, axis=0).reshape(1, DD) * 0.125, (8, DD))
    ssum = cs[0] + cs[1] + cs[2]
    for g in range(GN):
        uab_r[g] = _dot(cs[g], wd1) * dinvs[g]
        uab_r[GN + g] = _dot((ssum - cs[g]) * 0.5, wd1) * dinvs[g]
    m = ssum * (1.0 / GN)
    used_r[...] = jnp.where(m > 0, m, jnp.exp(jnp.minimum(m, 0.0)) - 1.0)


def _tc_cd(s2, u2, dinv, bc2, wd1):
    g3 = pl.BlockSpec((GN, RB, DD), lambda rb: (0, rb, 0))
    return pl.pallas_call(
        _tc_cd_body,
        grid=(NBLK,),
        in_specs=[
            pl.BlockSpec((GN, NC, RB, DD), lambda rb: (0, 0, rb, 0)),
            g3, _DINV_ALL, _B_FULL, _W_FULL,
        ],
        out_specs=[
            g3,
            pl.BlockSpec((2 * GN, RB, DD), lambda rb: (0, rb, 0)),
            pl.BlockSpec((RB, DD), lambda rb: (rb, 0)),
            pl.BlockSpec((1, GN, 8, DD), lambda rb: (rb, 0, 0, 0)),
        ],
        out_shape=[
            jax.ShapeDtypeStruct((GN, NN, DD), _F32),
            jax.ShapeDtypeStruct((2 * GN, NN, DD), _F32),
            jax.ShapeDtypeStruct((NN, DD), _F32),
            jax.ShapeDtypeStruct((NBLK, GN, 8, DD), _F32),
        ],
    )(s2, u2, dinv, bc2, wd1)


def _post6h_body(s_r, u_r, dinv_r, b_r, w2_r, ab1_r, u2_r):
    w2 = w2_r[...]
    for k in range(GN):
        dinv = dinv_r[k][:, 0:1]
        ab = (s_r[k, 0] + s_r[k, 1] + u_r[0, k]) * dinv + b_r[...]
        ab1_r[k] = ab
        u2_r[k] = _dot(ab, w2) * dinv


def _make_post6h(off3):
    g3 = pl.BlockSpec((GN, RB, DD), lambda rb: (0, rb, 0))

    def run(s_half, u_full, dinv, bias, w2):
        return pl.pallas_call(
            _post6h_body,
            grid=(NBLK,),
            in_specs=[
                pl.BlockSpec((GN, NC, RB, DD), lambda rb: (0, 0, rb, 0)),
                pl.BlockSpec((1, GN, RB, DD),
                             lambda rb: (off3, 0, rb, 0)),
                _DINV_ALL, _B_FULL, _W_FULL,
            ],
            out_specs=[g3, g3],
            out_shape=[
                jax.ShapeDtypeStruct((GN, NN, DD), _F32),
                jax.ShapeDtypeStruct((GN, NN, DD), _F32),
            ],
        )(s_half, u_full, dinv, bias, w2)

    return run


_post6a = _make_post6h(0)
_post6b = _make_post6h(1)


def _tc_f_body(sa_r, sb_r, ua_r, ub_r, dinv_r, b_r, wa_r, wb_r, bf_r,
               a2_r, fin_r):
    wa = wa_r[...]
    wb = wb_r[...]
    for g in range(GN):
        dinv = dinv_r[g][:, 0:1]
        a2 = (sa_r[g, 0] + sa_r[g, 1] + ua_r[g]) * dinv + b_r[...]
        b2 = (sb_r[g, 0] + sb_r[g, 1] + ub_r[g]) * dinv + b_r[...]
        fin_r[g] = _dot(a2, wa) + _dot(b2, wb) + bf_r[...]
        a2_r[g] = a2


def _tc_f(sab2a, sab2b, uab2a, uab2b, dinv, bd2, wf2a, wf2b, bfc2):
    g3 = pl.BlockSpec((GN, RB, DD), lambda rb: (0, rb, 0))
    snc = pl.BlockSpec((GN, NC, RB, DD), lambda rb: (0, 0, rb, 0))
    return pl.pallas_call(
        _tc_f_body,
        grid=(NBLK,),
        in_specs=[
            snc, snc, g3, g3,
            _DINV_ALL, _B_FULL, _W_FULL, _W_FULL, _B_FULL,
        ],
        out_specs=[g3, g3],
        out_shape=[
            jax.ShapeDtypeStruct((GN, NN, DD), _F32),
            jax.ShapeDtypeStruct((GN, NN, DD), _F32),
        ],
    )(sab2a, sab2b, uab2a, uab2b, dinv, bd2, wf2a, wf2b, bfc2)


def _tc_g_body(fin_r, c2_r, h0_r, a2_r, c1_r, ab1_r, csp_r, wmt_r,
               fuse_r, comp_r, obfp_r):
    fin = fin_r[...]
    c2 = c2_r[...]
    csum = jnp.sum(csp_r[...], axis=(0, 2))
    gv = jax.nn.sigmoid(csum * (1.0 / NN))
    v = [_dot(gv[i][None, :], wmt_r[i])[0] for i in range(GN)]
    fuse_r[...] = jax.nn.sigmoid((fin[0] + fin[1] + fin[2]) * (1.0 / GN))
    cols = []
    for i in range(GN):
        ls = []
        for j in range(GN):
            ls.append(jax.nn.sigmoid(
                jnp.sum(c2[j] * v[i][None, :], axis=1, keepdims=True)))
        lc = jnp.concatenate(ls, axis=1)
        cols.append(lc / jnp.sum(lc, axis=1, keepdims=True))
    comp_r[...] = jnp.concatenate(
        cols + [jnp.zeros((RB, LN - GN * GN), _F32)], axis=1)
    h0 = h0_r[...]
    a2 = a2_r[...]
    c1 = c1_r[...]
    ab1 = ab1_r[...]
    parts = []
    for i in range(GN):
        parts.append(jnp.sum((h0[i] - a2[i]) ** 2))
    for i in range(GN):
        parts.append(jnp.sum((c1[i] - ab1[i]) ** 2))
    for (i, j) in ((0, 1), (0, 2), (1, 2)):
        parts.append(jnp.sum((fin[i] - fin[j]) ** 2))
    row = jnp.concatenate(
        [p.reshape(1, 1, 1) for p in parts] +
        [jnp.zeros((1, 1, LN - len(parts)), _F32)], axis=2)
    obfp_r[...] = jnp.broadcast_to(row * 0.125, (1, 8, LN))


def _tc_g(fin, c2, h0, a2, c1, ab1, csp, wmixt):
    blk3 = pl.BlockSpec((GN, RB, DD), lambda rb: (0, rb, 0))
    return pl.pallas_call(
        _tc_g_body,
        grid=(NBLK,),
        in_specs=[
            blk3, blk3, blk3, blk3, blk3, blk3,
            pl.BlockSpec((NBLK, GN, 8, DD), lambda rb: (0, 0, 0, 0)),
            pl.BlockSpec((GN, DD, DD), lambda rb: (0, 0, 0)),
        ],
        out_specs=[
            pl.BlockSpec((RB, DD), lambda rb: (rb, 0)),
            pl.BlockSpec((RB, LN), lambda rb: (rb, 0)),
            pl.BlockSpec((1, 8, LN), lambda rb: (rb, 0, 0)),
        ],
        out_shape=[
            jax.ShapeDtypeStruct((NN, DD), _F32),
            jax.ShapeDtypeStruct((NN, LN), _F32),
            jax.ShapeDtypeStruct((NBLK, 8, LN), _F32),
        ],
    )(fin, c2, h0, a2, c1, ab1, csp, wmixt)


def _tc_h_body(obfp_r, o1_r, o2_r):
    tot = jnp.sum(obfp_r[...].reshape(NBLK * 8, LN), axis=0,
                  keepdims=True)
    obf1 = jnp.float32(0.0)
    for i in range(GN):
        obf1 = obf1 + (jnp.sqrt(tot[0, i]) + jnp.sqrt(tot[0, GN + i])) * 0.5
    obf0 = 2.0 * (jnp.sqrt(tot[0, 6]) + jnp.sqrt(tot[0, 7])
                  + jnp.sqrt(tot[0, 8]))
    o1_r[...] = jnp.broadcast_to(obf1, (1, DD))
    o2_r[...] = jnp.broadcast_to(obf0, (1, DD))


def _tc_h(obfp):
    return pl.pallas_call(
        _tc_h_body,
        grid=(1,),
        in_specs=[pl.BlockSpec((NBLK, 8, LN), lambda i: (0, 0, 0))],
        out_specs=[
            pl.BlockSpec((1, DD), lambda i: (0, 0)),
            pl.BlockSpec((1, DD), lambda i: (0, 0)),
        ],
        out_shape=[
            jax.ShapeDtypeStruct((1, DD), _F32),
            jax.ShapeDtypeStruct((1, DD), _F32),
        ],
    )(obfp)


def kernel(x0, x1, x2, ei0, ei1, ei2, W_fc1, b_fc1, Wc1, bc1, Wc2, bc2,
           Wd1, bd1, Wd2, bd2, W_fc2, b_fc2, Wmix):
    pad = EPAD - EE
    ar = jnp.arange(pad, dtype=jnp.int32)
    psrc = (ar * 97) % NN
    pdst = NN + (ar % NSINK)
    srcs = []
    dsts = []
    for ei in (ei0, ei1, ei2):
        srcs.append(jnp.concatenate([ei[0], psrc]).reshape(ROWS2D, CHUNK))
        dsts.append(jnp.concatenate([ei[1], pdst]).reshape(ROWS2D, CHUNK))

    z128 = jnp.zeros((NA, DD), _F32)
    ones128 = jnp.broadcast_to(
        (jnp.arange(DD) == 0).astype(_F32), (CHUNK, DD))

    deg = _deg(z128, ones128, dsts[0], dsts[1], dsts[2])
    dinv = _tc_dinv(deg)

    h0, xw1 = _tc_a(x0, x1, x2, W_fc1, b_fc1.reshape(GN, 1, DD), Wc1)
    u1 = _tc_scale(xw1, dinv)
    s1 = _prop3(z128, u1, srcs[0], srcs[1], srcs[2],
                dsts[0], dsts[1], dsts[2])
    c1, u2 = _post3_mm(s1, u1, dinv, bc1.reshape(1, DD), Wc2)
    s2 = _prop3(z128, u2, srcs[0], srcs[1], srcs[2],
                dsts[0], dsts[1], dsts[2])
    c2, uab1, used, csp = _tc_cd(s2, u2, dinv, bc2.reshape(1, DD), Wd1)
    ea = (srcs[0], srcs[1], srcs[2], dsts[0], dsts[1], dsts[2])
    uab1r = uab1.reshape(NC, GN, NN, DD)
    sab1a = _prop3a(z128, uab1, *ea)
    sab1b = _prop3b(z128, uab1, *ea)
    ab1, uab2a = _post6a(sab1a, uab1r, dinv, bd1.reshape(1, DD), Wd2)
    _, uab2b = _post6b(sab1b, uab1r, dinv, bd1.reshape(1, DD), Wd2)
    sab2a = _prop3(z128, uab2a, *ea)
    sab2b = _prop3(z128, uab2b, *ea)
    a2, fin = _tc_f(sab2a, sab2b, uab2a, uab2b, dinv, bd2.reshape(1, DD),
                    W_fc2[:DD], W_fc2[DD:], b_fc2.reshape(1, DD))
    fuse, comp16, obfp = _tc_g(fin, c2, h0, a2, c1, ab1, csp,
                               Wmix.transpose(0, 2, 1))
    o1, o2 = _tc_h(obfp)
    return (fuse, used, comp16[:, :GN * GN], o1[0, 0], o2[0, 0])


# speedup vs baseline: 1.0061x; 1.0061x over previous
import functools

import jax
import jax.numpy as jnp
from jax import lax
from jax.experimental import pallas as pl
from jax.experimental.pallas import tpu as pltpu
from jax.experimental.pallas import tpu_sc as plsc

GN = 3
NN = 10000
DD = 128
EE = 320000

NC = 2
NS = 16
NW = NC * NS
LN = 16

CHUNK = 64
CPB = 40
EPW = 160 * CHUNK
EPAD = EPW * NW
ROWS2D = EPAD // CHUNK
NSINK = 112
NA = NN + NSINK
RZ = NA // NS
RF = 624
RTAIL = NN - NS * RF

NBLK = 10
RB = NN // NBLK

_MESH = plsc.VectorSubcoreMesh(core_axis_name="c", subcore_axis_name="s",
                               num_cores=NC, num_subcores=NS)
_F32 = jnp.float32


@functools.partial(
    pl.kernel,
    out_type=jax.ShapeDtypeStruct((GN, NC, NN, DD), _F32),
    mesh=_MESH,
    scratch_types=[
        pltpu.VMEM_SHARED((NA, DD), _F32),
        pltpu.VMEM((CPB, CHUNK), jnp.int32),
        pltpu.VMEM((CHUNK, DD), _F32),
        pltpu.SemaphoreType.DMA,
    ],
)
def _deg(zeros_h, ones_h, d0, d1, d2, out, acc, didx, ones_b, ssem):
    c = lax.axis_index("c")
    s = lax.axis_index("s")
    w = s * NC + c
    pltpu.sync_copy(ones_h, ones_b)

    nblocks = EPW // CHUNK // CPB
    for g, dref in enumerate((d0, d1, d2)):
        pltpu.sync_copy(zeros_h.at[pl.ds(s * RZ, RZ)],
                        acc.at[pl.ds(s * RZ, RZ)])
        plsc.subcore_barrier()
        base = w * (EPW // CHUNK)

        @pl.loop(0, nblocks)
        def _(b):
            r0 = base + b * CPB
            pltpu.sync_copy(dref.at[pl.ds(r0, CPB)], didx)
            descs = [pltpu.async_copy(ones_b, acc.at[didx.at[j]], ssem,
                                      add=True)
                     for j in range(CPB)]
            for d in descs:
                d.wait()

        plsc.subcore_barrier()
        pltpu.sync_copy(acc.at[pl.ds(s * RF, RF)],
                        out.at[g, c, pl.ds(s * RF, RF)])

        @pl.when(s == NS - 1)
        def _():
            pltpu.sync_copy(acc.at[pl.ds(NS * RF, RTAIL)],
                            out.at[g, c, pl.ds(NS * RF, RTAIL)])

        plsc.subcore_barrier()


def _make_prop(kk, gids, off=0, ntab=None):
    if ntab is None:
        ntab = kk

    @functools.partial(
        pl.kernel,
        out_type=jax.ShapeDtypeStruct((kk, NC, NN, DD), _F32),
        mesh=_MESH,
        scratch_types=[
            pltpu.VMEM_SHARED((NA, DD), _F32),
            pltpu.VMEM((CPB, CHUNK), jnp.int32),
            pltpu.VMEM((CPB, CHUNK), jnp.int32),
            pltpu.VMEM((CHUNK, DD), _F32),
            pltpu.VMEM((CHUNK, DD), _F32),
            pltpu.VMEM((CHUNK, DD), _F32),
            pltpu.VMEM((CHUNK, DD), _F32),
            pltpu.SemaphoreType.DMA,
            pltpu.SemaphoreType.DMA,
            pltpu.SemaphoreType.DMA,
            pltpu.SemaphoreType.DMA,
            pltpu.SemaphoreType.DMA,
            pltpu.SemaphoreType.DMA,
            pltpu.SemaphoreType.DMA,
            pltpu.SemaphoreType.DMA,
        ],
    )
    def prop(zeros_h, tabs, s0, s1, s2, d0, d1, d2, out,
             acc, sidx, didx, r0b, r1b, r2b, r3b,
             gs0, gs1, gs2, gs3, ss0, ss1, ss2, ss3):
        c = lax.axis_index("c")
        s = lax.axis_index("s")
        w = s * NC + c

        rows = (r0b, r1b, r2b, r3b)
        gsem = (gs0, gs1, gs2, gs3)
        ssem = (ss0, ss1, ss2, ss3)
        srcs = (s0, s1, s2)
        dsts = (d0, d1, d2)
        nblocks = EPW // CHUNK // CPB
        for k in range(kk):
            g = gids[k]
            pltpu.sync_copy(zeros_h.at[pl.ds(s * RZ, RZ)],
                            acc.at[pl.ds(s * RZ, RZ)])
            plsc.subcore_barrier()
            base = w * (EPW // CHUNK)
            tab = tabs.at[off + k]

            @pl.loop(0, nblocks)
            def _(b):
                r0 = base + b * CPB
                pltpu.sync_copy(srcs[g].at[pl.ds(r0, CPB)], sidx)
                pltpu.sync_copy(dsts[g].at[pl.ds(r0, CPB)], didx)
                gd = [None] * CPB
                sd = [None] * CPB
                gd[0] = pltpu.async_copy(tab.at[sidx.at[0]], rows[0],
                                         gsem[0])
                gd[1] = pltpu.async_copy(tab.at[sidx.at[1]], rows[1],
                                         gsem[1])
                for j in range(CPB):
                    p = j % 4
                    gd[j].wait()
                    sd[j] = pltpu.async_copy(rows[p], acc.at[didx.at[j]],
                                             ssem[p], add=True)
                    if j + 2 < CPB:
                        if j >= 2:
                            sd[j - 2].wait()
                        q = (j + 2) % 4
                        gd[j + 2] = pltpu.async_copy(
                            tab.at[sidx.at[j + 2]], rows[q], gsem[q])
                for j in range(CPB - 4, CPB):
                    sd[j].wait()

            plsc.subcore_barrier()
            pltpu.sync_copy(acc.at[pl.ds(s * RF, RF)],
                            out.at[k, c, pl.ds(s * RF, RF)])

            @pl.when(s == NS - 1)
            def _():
                pltpu.sync_copy(acc.at[pl.ds(NS * RF, RTAIL)],
                                out.at[k, c, pl.ds(NS * RF, RTAIL)])

            plsc.subcore_barrier()

    return prop


_prop3 = _make_prop(3, (0, 1, 2))
_prop3a = _make_prop(3, (0, 1, 2), off=0, ntab=6)
_prop3b = _make_prop(3, (0, 1, 2), off=3, ntab=6)
_prop2_01 = _make_prop(2, (0, 1), off=0, ntab=3)
_prop1_2 = _make_prop(1, (2,), off=2, ntab=3)
_prop2s = _make_prop(2, (0, 1))
_prop1s = _make_prop(1, (2,))


def _dot(a, b):
    return jnp.dot(a, b, preferred_element_type=_F32,
                   precision=lax.Precision.HIGHEST)


def _tc_dinv_body(deg_r, dinv_r):
    dd = deg_r[0]
    cnt = dd[0, :, 0:1] + dd[1, :, 0:1] + 1.0
    dinv = lax.rsqrt(jnp.maximum(cnt, 1e-12))
    dinv_r[0] = jnp.broadcast_to(dinv, (RB, LN))


def _tc_dinv(deg):
    return pl.pallas_call(
        _tc_dinv_body,
        grid=(GN, NBLK),
        in_specs=[pl.BlockSpec((1, NC, RB, DD), lambda g, rb: (g, 0, rb, 0))],
        out_specs=pl.BlockSpec((1, RB, LN), lambda g, rb: (g, rb, 0)),
        out_shape=jax.ShapeDtypeStruct((GN, NN, LN), _F32),
    )(deg)


def _tc_a_body(x0_r, x1_r, x2_r, w_r, b_r, wc_r, h_r, xw_r):
    wc = wc_r[...]
    for g, x_r in enumerate((x0_r, x1_r, x2_r)):
        h = _dot(x_r[...], w_r[g]) + b_r[g]
        xw_r[g] = _dot(h, wc)
        h_r[g] = h


_DINV_ALL = pl.BlockSpec((GN, RB, LN), lambda rb: (0, rb, 0))
_W_FULL = pl.BlockSpec((DD, DD), lambda rb: (0, 0))
_B_FULL = pl.BlockSpec((1, DD), lambda rb: (0, 0))


def _tc_a(x0, x1, x2, wfc1, bfc1, wc1):
    xspec = pl.BlockSpec((RB, DD), lambda rb: (rb, 0))
    g3 = pl.BlockSpec((GN, RB, DD), lambda rb: (0, rb, 0))
    return pl.pallas_call(
        _tc_a_body,
        grid=(NBLK,),
        in_specs=[
            xspec, xspec, xspec,
            pl.BlockSpec((GN, DD, DD), lambda rb: (0, 0, 0)),
            pl.BlockSpec((GN, 1, DD), lambda rb: (0, 0, 0)),
            _W_FULL,
        ],
        out_specs=[g3, g3],
        out_shape=[
            jax.ShapeDtypeStruct((GN, NN, DD), _F32),
            jax.ShapeDtypeStruct((GN, NN, DD), _F32),
        ],
    )(x0, x1, x2, wfc1, bfc1, wc1)


def _tc_scale_body(xw_r, dinv_r, u_r):
    for g in range(GN):
        u_r[g] = xw_r[g] * dinv_r[g][:, 0:1]


def _tc_scale(xw, dinv):
    g3 = pl.BlockSpec((GN, RB, DD), lambda rb: (0, rb, 0))
    return pl.pallas_call(
        _tc_scale_body,
        grid=(NBLK,),
        in_specs=[g3, _DINV_ALL],
        out_specs=g3,
        out_shape=jax.ShapeDtypeStruct((GN, NN, DD), _F32),
    )(xw, dinv)


def _make_posth(goff, gcnt):
    def body(*refs):
        s_r = refs[0]
        u_rs = refs[1:1 + gcnt]
        dinv_rs = refs[1 + gcnt:1 + 2 * gcnt]
        b_r, w2_r, c_r, u2_r = refs[1 + 2 * gcnt:]
        w2 = w2_r[...]
        for g in range(gcnt):
            dinv = dinv_rs[g][0][:, 0:1]
            cfull = (s_r[g, 0] + s_r[g, 1] + u_rs[g][0]) * dinv + b_r[...]
            c_r[g] = cfull
            u2_r[g] = _dot(cfull, w2) * dinv

    gc = pl.BlockSpec((gcnt, RB, DD), lambda rb: (0, rb, 0))

    def run(s_arr, u_full, dinv, bias, w2):
        in_specs = [pl.BlockSpec((gcnt, NC, RB, DD),
                                 lambda rb: (0, 0, rb, 0))]
        args = [s_arr]
        for g in range(gcnt):
            in_specs.append(pl.BlockSpec(
                (1, RB, DD), lambda rb, _g=g: (goff + _g, rb, 0)))
            args.append(u_full)
        for g in range(gcnt):
            in_specs.append(pl.BlockSpec(
                (1, RB, LN), lambda rb, _g=g: (goff + _g, rb, 0)))
            args.append(dinv)
        in_specs += [_B_FULL, _W_FULL]
        args += [bias, w2]
        return pl.pallas_call(
            body,
            grid=(NBLK,),
            in_specs=in_specs,
            out_specs=[gc, gc],
            out_shape=[
                jax.ShapeDtypeStruct((gcnt, NN, DD), _F32),
                jax.ShapeDtypeStruct((gcnt, NN, DD), _F32),
            ],
        )(*args)

    return run


_post01 = _make_posth(0, 2)
_post2g = _make_posth(2, 1)


def _tc_cd_body(sa_r, sb_r, ua_r, ub_r, dinv_r, b_r, wd1_r,
                c_r, uab_r, used_r, csp_r):
    wd1 = wd1_r[...]
    cs = []
    dinvs = []
    for g in range(GN):
        s_g = sa_r[g] if g < 2 else sb_r[0]
        u_g = ua_r[g] if g < 2 else ub_r[0]
        dinv = dinv_r[g][:, 0:1]
        dinvs.append(dinv)
        cfull = (s_g[0] + s_g[1] + u_g) * dinv + b_r[...]
        c_r[g] = cfull
        cs.append(cfull)
        csp_r[0, g] = jnp.broadcast_to(
            jnp.sum(cfull, axis=0).reshape(1, DD) * 0.125, (8, DD))
    ssum = cs[0] + cs[1] + cs[2]
    for g in range(GN):
        uab_r[g] = _dot(cs[g], wd1) * dinvs[g]
        uab_r[GN + g] = _dot((ssum - cs[g]) * 0.5, wd1) * dinvs[g]
    m = ssum * (1.0 / GN)
    used_r[...] = jnp.where(m > 0, m, jnp.exp(jnp.minimum(m, 0.0)) - 1.0)


def _tc_cd(s2a, s2b, u2a, u2b, dinv, bc2, wd1):
    g3 = pl.BlockSpec((GN, RB, DD), lambda rb: (0, rb, 0))
    return pl.pallas_call(
        _tc_cd_body,
        grid=(NBLK,),
        in_specs=[
            pl.BlockSpec((2, NC, RB, DD), lambda rb: (0, 0, rb, 0)),
            pl.BlockSpec((1, NC, RB, DD), lambda rb: (0, 0, rb, 0)),
            pl.BlockSpec((2, RB, DD), lambda rb: (0, rb, 0)),
            pl.BlockSpec((1, RB, DD), lambda rb: (0, rb, 0)),
            _DINV_ALL, _B_FULL, _W_FULL,
        ],
        out_specs=[
            g3,
            pl.BlockSpec((2 * GN, RB, DD), lambda rb: (0, rb, 0)),
            pl.BlockSpec((RB, DD), lambda rb: (rb, 0)),
            pl.BlockSpec((1, GN, 8, DD), lambda rb: (rb, 0, 0, 0)),
        ],
        out_shape=[
            jax.ShapeDtypeStruct((GN, NN, DD), _F32),
            jax.ShapeDtypeStruct((2 * GN, NN, DD), _F32),
            jax.ShapeDtypeStruct((NN, DD), _F32),
            jax.ShapeDtypeStruct((NBLK, GN, 8, DD), _F32),
        ],
    )(s2a, s2b, u2a, u2b, dinv, bc2, wd1)


def _post6h_body(s_r, u_r, dinv_r, b_r, w2_r, ab1_r, u2_r):
    w2 = w2_r[...]
    for k in range(GN):
        dinv = dinv_r[k][:, 0:1]
        ab = (s_r[k, 0] + s_r[k, 1] + u_r[0, k]) * dinv + b_r[...]
        ab1_r[k] = ab
        u2_r[k] = _dot(ab, w2) * dinv


def _make_post6h(off3):
    g3 = pl.BlockSpec((GN, RB, DD), lambda rb: (0, rb, 0))

    def run(s_half, u_full, dinv, bias, w2):
        return pl.pallas_call(
            _post6h_body,
            grid=(NBLK,),
            in_specs=[
                pl.BlockSpec((GN, NC, RB, DD), lambda rb: (0, 0, rb, 0)),
                pl.BlockSpec((1, GN, RB, DD),
                             lambda rb: (off3, 0, rb, 0)),
                _DINV_ALL, _B_FULL, _W_FULL,
            ],
            out_specs=[g3, g3],
            out_shape=[
                jax.ShapeDtypeStruct((GN, NN, DD), _F32),
                jax.ShapeDtypeStruct((GN, NN, DD), _F32),
            ],
        )(s_half, u_full, dinv, bias, w2)

    return run


_post6a = _make_post6h(0)
_post6b = _make_post6h(1)


def _tc_f_body(sa_r, sb_r, ua_r, ub_r, dinv_r, b_r, wa_r, wb_r, bf_r,
               a2_r, fin_r):
    wa = wa_r[...]
    wb = wb_r[...]
    for g in range(GN):
        dinv = dinv_r[g][:, 0:1]
        a2 = (sa_r[g, 0] + sa_r[g, 1] + ua_r[g]) * dinv + b_r[...]
        b2 = (sb_r[g, 0] + sb_r[g, 1] + ub_r[g]) * dinv + b_r[...]
        fin_r[g] = _dot(a2, wa) + _dot(b2, wb) + bf_r[...]
        a2_r[g] = a2


def _tc_f(sab2a, sab2b, uab2a, uab2b, dinv, bd2, wf2a, wf2b, bfc2):
    g3 = pl.BlockSpec((GN, RB, DD), lambda rb: (0, rb, 0))
    snc = pl.BlockSpec((GN, NC, RB, DD), lambda rb: (0, 0, rb, 0))
    return pl.pallas_call(
        _tc_f_body,
        grid=(NBLK,),
        in_specs=[
            snc, snc, g3, g3,
            _DINV_ALL, _B_FULL, _W_FULL, _W_FULL, _B_FULL,
        ],
        out_specs=[g3, g3],
        out_shape=[
            jax.ShapeDtypeStruct((GN, NN, DD), _F32),
            jax.ShapeDtypeStruct((GN, NN, DD), _F32),
        ],
    )(sab2a, sab2b, uab2a, uab2b, dinv, bd2, wf2a, wf2b, bfc2)


def _tc_g_body(fin_r, c2_r, h0_r, a2_r, c1a_r, c1b_r, ab1_r, csp_r, wmt_r,
               fuse_r, comp_r, obfp_r):
    fin = fin_r[...]
    c2 = c2_r[...]
    csum = jnp.sum(csp_r[...], axis=(0, 2))
    gv = jax.nn.sigmoid(csum * (1.0 / NN))
    v = [_dot(gv[i][None, :], wmt_r[i])[0] for i in range(GN)]
    fuse_r[...] = jax.nn.sigmoid((fin[0] + fin[1] + fin[2]) * (1.0 / GN))
    cols = []
    for i in range(GN):
        ls = []
        for j in range(GN):
            ls.append(jax.nn.sigmoid(
                jnp.sum(c2[j] * v[i][None, :], axis=1, keepdims=True)))
        lc = jnp.concatenate(ls, axis=1)
        cols.append(lc / jnp.sum(lc, axis=1, keepdims=True))
    comp_r[...] = jnp.concatenate(
        cols + [jnp.zeros((RB, LN - GN * GN), _F32)], axis=1)
    h0 = h0_r[...]
    a2 = a2_r[...]
    c1 = [c1a_r[0], c1a_r[1], c1b_r[0]]
    ab1 = ab1_r[...]
    parts = []
    for i in range(GN):
        parts.append(jnp.sum((h0[i] - a2[i]) ** 2))
    for i in range(GN):
        parts.append(jnp.sum((c1[i] - ab1[i]) ** 2))
    for (i, j) in ((0, 1), (0, 2), (1, 2)):
        parts.append(jnp.sum((fin[i] - fin[j]) ** 2))
    row = jnp.concatenate(
        [p.reshape(1, 1, 1) for p in parts] +
        [jnp.zeros((1, 1, LN - len(parts)), _F32)], axis=2)
    obfp_r[...] = jnp.broadcast_to(row * 0.125, (1, 8, LN))


def _tc_g(fin, c2, h0, a2, c1a, c1b, ab1, csp, wmixt):
    blk3 = pl.BlockSpec((GN, RB, DD), lambda rb: (0, rb, 0))
    return pl.pallas_call(
        _tc_g_body,
        grid=(NBLK,),
        in_specs=[
            blk3, blk3, blk3, blk3,
            pl.BlockSpec((2, RB, DD), lambda rb: (0, rb, 0)),
            pl.BlockSpec((1, RB, DD), lambda rb: (0, rb, 0)),
            blk3,
            pl.BlockSpec((NBLK, GN, 8, DD), lambda rb: (0, 0, 0, 0)),
            pl.BlockSpec((GN, DD, DD), lambda rb: (0, 0, 0)),
        ],
        out_specs=[
            pl.BlockSpec((RB, DD), lambda rb: (rb, 0)),
            pl.BlockSpec((RB, LN), lambda rb: (rb, 0)),
            pl.BlockSpec((1, 8, LN), lambda rb: (rb, 0, 0)),
        ],
        out_shape=[
            jax.ShapeDtypeStruct((NN, DD), _F32),
            jax.ShapeDtypeStruct((NN, LN), _F32),
            jax.ShapeDtypeStruct((NBLK, 8, LN), _F32),
        ],
    )(fin, c2, h0, a2, c1a, c1b, ab1, csp, wmixt)


def _tc_h_body(obfp_r, o1_r, o2_r):
    tot = jnp.sum(obfp_r[...].reshape(NBLK * 8, LN), axis=0,
                  keepdims=True)
    obf1 = jnp.float32(0.0)
    for i in range(GN):
        obf1 = obf1 + (jnp.sqrt(tot[0, i]) + jnp.sqrt(tot[0, GN + i])) * 0.5
    obf0 = 2.0 * (jnp.sqrt(tot[0, 6]) + jnp.sqrt(tot[0, 7])
                  + jnp.sqrt(tot[0, 8]))
    o1_r[...] = jnp.broadcast_to(obf1, (1, DD))
    o2_r[...] = jnp.broadcast_to(obf0, (1, DD))


def _tc_h(obfp):
    return pl.pallas_call(
        _tc_h_body,
        grid=(1,),
        in_specs=[pl.BlockSpec((NBLK, 8, LN), lambda i: (0, 0, 0))],
        out_specs=[
            pl.BlockSpec((1, DD), lambda i: (0, 0)),
            pl.BlockSpec((1, DD), lambda i: (0, 0)),
        ],
        out_shape=[
            jax.ShapeDtypeStruct((1, DD), _F32),
            jax.ShapeDtypeStruct((1, DD), _F32),
        ],
    )(obfp)


def kernel(x0, x1, x2, ei0, ei1, ei2, W_fc1, b_fc1, Wc1, bc1, Wc2, bc2,
           Wd1, bd1, Wd2, bd2, W_fc2, b_fc2, Wmix):
    pad = EPAD - EE
    ar = jnp.arange(pad, dtype=jnp.int32)
    psrc = (ar * 97) % NN
    pdst = NN + (ar % NSINK)
    srcs = []
    dsts = []
    for ei in (ei0, ei1, ei2):
        srcs.append(jnp.concatenate([ei[0], psrc]).reshape(ROWS2D, CHUNK))
        dsts.append(jnp.concatenate([ei[1], pdst]).reshape(ROWS2D, CHUNK))

    z128 = jnp.zeros((NA, DD), _F32)
    ones128 = jnp.broadcast_to(
        (jnp.arange(DD) == 0).astype(_F32), (CHUNK, DD))

    deg = _deg(z128, ones128, dsts[0], dsts[1], dsts[2])
    dinv = _tc_dinv(deg)

    h0, xw1 = _tc_a(x0, x1, x2, W_fc1, b_fc1.reshape(GN, 1, DD), Wc1)
    u1 = _tc_scale(xw1, dinv)
    ea = (srcs[0], srcs[1], srcs[2], dsts[0], dsts[1], dsts[2])
    s1a = _prop2_01(z128, u1, *ea)
    s1b = _prop1_2(z128, u1, *ea)
    c1a, u2a = _post01(s1a, u1, dinv, bc1.reshape(1, DD), Wc2)
    c1b, u2b = _post2g(s1b, u1, dinv, bc1.reshape(1, DD), Wc2)
    s2a = _prop2s(z128, u2a, *ea)
    s2b = _prop1s(z128, u2b, *ea)
    c2, uab1, used, csp = _tc_cd(s2a, s2b, u2a, u2b, dinv,
                                 bc2.reshape(1, DD), Wd1)
    uab1r = uab1.reshape(2, GN, NN, DD)
    sab1a = _prop3a(z128, uab1, *ea)
    sab1b = _prop3b(z128, uab1, *ea)
    ab1, uab2a = _post6a(sab1a, uab1r, dinv, bd1.reshape(1, DD), Wd2)
    _, uab2b = _post6b(sab1b, uab1r, dinv, bd1.reshape(1, DD), Wd2)
    sab2a = _prop3(z128, uab2a, *ea)
    sab2b = _prop3(z128, uab2b, *ea)
    a2, fin = _tc_f(sab2a, sab2b, uab2a, uab2b, dinv, bd2.reshape(1, DD),
                    W_fc2[:DD], W_fc2[DD:], b_fc2.reshape(1, DD))
    fuse, comp16, obfp = _tc_g(fin, c2, h0, a2, c1a, c1b, ab1, csp,
                               Wmix.transpose(0, 2, 1))
    o1, o2 = _tc_h(obfp)
    return (fuse, used, comp16[:, :GN * GN], o1[0, 0], o2[0, 0])
